# baseline (speedup 1.0000x reference)
import functools

import jax
import jax.numpy as jnp
from jax import lax
from jax.experimental import pallas as pl
from jax.experimental.pallas import tpu as pltpu
from jax.experimental.pallas import tpu_sc as plsc

N_NODES = 10000
DIM = 128
N_EDGES = 320000
BATCH = 4096

NUM_CORES = 2
NUM_SUBCORES = 16
NW = NUM_CORES * NUM_SUBCORES

ECHUNK = 128
ESUB = ECHUNK // 128
NCHUNKS = N_EDGES // ECHUNK
_EITERS_CEIL = -(-NCHUNKS // NW)
EITERS = 12 * (-(-_EITERS_CEIL // 12))

SLAB = 80
NSLABS = N_NODES // SLAB
SITERS = -(-NSLABS // NUM_SUBCORES)

_mesh = plsc.VectorSubcoreMesh(core_axis_name="c", subcore_axis_name="s")


def _vec_add_rows(dst_ref, src_ref, nrows):
    def body(r, carry):
        for d in range(DIM // 16):
            sl = pl.ds(d * 16, 16)
            dst_ref[r, sl] = dst_ref[r, sl] + src_ref[r, sl]
        return carry
    lax.fori_loop(0, nrows, body, 0)


@functools.partial(
    pl.kernel,
    out_type=jax.ShapeDtypeStruct((NUM_CORES, N_NODES, DIM), jnp.float32),
    mesh=_mesh,
    scratch_types=[
        [pltpu.VMEM((ESUB, 128), jnp.int32)] * 4,
        [pltpu.VMEM((ESUB, 128), jnp.int32)] * 4,
        [pltpu.VMEM((ECHUNK,), jnp.float32)] * 4,
        [pltpu.VMEM((ECHUNK, DIM), jnp.float32)] * 3,
        pltpu.VMEM_SHARED((N_NODES, DIM), jnp.float32),
        [pltpu.SemaphoreType.DMA] * 4,
        [pltpu.SemaphoreType.DMA] * 4,
        [pltpu.SemaphoreType.DMA] * 3,
        [pltpu.SemaphoreType.DMA] * 2,
    ],
)
def _prop(table, srcs, dsts, ews, out, src_v, dst_v, ew_v, rows_v,
          acc, semi, semd, semg, sems):
    c = lax.axis_index("c")
    s = lax.axis_index("s")
    wid = s * NUM_CORES + c

    zero16 = jnp.zeros((16,), jnp.float32)

    def zrow(r, carry):
        for d in range(DIM // 16):
            rows_v[0][r, pl.ds(d * 16, 16)] = zero16
        return carry
    lax.fori_loop(0, ECHUNK, zrow, 0)

    def zslab(k, carry):
        sid = k * NUM_SUBCORES + s

        @pl.when(sid < NSLABS)
        def _():
            pltpu.sync_copy(rows_v[0].at[pl.ds(0, SLAB)],
                            acc.at[pl.ds(sid * SLAB, SLAB)])
        return carry
    lax.fori_loop(0, SITERS, zslab, 0)
    plsc.subcore_barrier()

    nchunks_j = jnp.int32(NCHUNKS)

    def valid(i):
        return i * NW + wid < nchunks_j

    def issue_srcew(i, r):
        cid = jnp.minimum(i * NW + wid, nchunks_j - 1)
        pltpu.async_copy(srcs.at[cid], src_v[r], semi[r])
        pltpu.async_copy(ews.at[cid], ew_v[r], semi[r])

    def wait_srcew(r):
        pltpu.make_async_copy(srcs.at[0], src_v[r], semi[r]).wait()
        pltpu.make_async_copy(ews.at[0], ew_v[r], semi[r]).wait()

    def issue_dst(i, r):
        cid = jnp.minimum(i * NW + wid, nchunks_j - 1)
        pltpu.async_copy(dsts.at[cid], dst_v[r], semd[r])

    def wait_dst(r):
        pltpu.make_async_copy(dsts.at[0], dst_v[r], semd[r]).wait()

    def issue_gather(r4, r3):
        for j in range(ESUB):
            pltpu.async_copy(table.at[src_v[r4].at[j]],
                             rows_v[r3].at[pl.ds(j * 128, 128)], semg[r3])

    def wait_gather(r4, r3):
        for j in range(ESUB):
            pltpu.make_async_copy(table.at[src_v[r4].at[j]],
                                  rows_v[r3].at[pl.ds(j * 128, 128)],
                                  semg[r3]).wait()

    def issue_scatter(r4, r3, b2):
        for j in range(ESUB):
            pltpu.async_copy(rows_v[r3].at[pl.ds(j * 128, 128)],
                             acc.at[dst_v[r4].at[j]], sems[b2], add=True)

    def wait_scatter(r4, r3, b2):
        for j in range(ESUB):
            pltpu.make_async_copy(rows_v[r3].at[pl.ds(j * 128, 128)],
                                  acc.at[dst_v[r4].at[j]], sems[b2]).wait()

    for r in range(3):
        issue_srcew(jnp.int32(r), r)
    for r in range(2):
        issue_dst(jnp.int32(r), r)
    wait_srcew(0)
    issue_gather(0, 0)
    wait_srcew(1)
    issue_gather(1, 1)

    def pipe(k, carry):
        for u in range(12):
            i12 = k * 12 + u
            r3 = u % 3
            r4 = u % 4
            b2 = u % 2
            wait_gather(r4, r3)

            @pl.when((i12 >= 1) & valid(i12 - 1))
            def _():
                wait_scatter((u - 1) % 4, (u - 1) % 3, 1 - b2)
            issue_dst(i12 + 2, (u + 2) % 4)
            issue_srcew(i12 + 3, (u + 3) % 4)
            wait_srcew((u + 2) % 4)
            issue_gather((u + 2) % 4, (u + 2) % 3)
            wait_dst(r4)

            @pl.when(valid(i12))
            def _():
                def scale(g, carry2):
                    wv = ew_v[r4][pl.ds(g * 16, 16)]
                    for j in range(16):
                        w = wv[j]
                        e = g * 16 + j
                        for d in range(DIM // 16):
                            sl = pl.ds(d * 16, 16)
                            rows_v[r3][e, sl] = rows_v[r3][e, sl] * w
                    return carry2
                lax.fori_loop(0, ECHUNK // 16, scale, 0)
                issue_scatter(r4, r3, b2)
        return carry
    lax.fori_loop(0, EITERS // 12, pipe, 0)
    wait_gather(EITERS % 4, EITERS % 3)
    wait_gather((EITERS + 1) % 4, (EITERS + 1) % 3)
    wait_srcew((EITERS + 2) % 4)
    wait_dst(EITERS % 4)
    wait_dst((EITERS + 1) % 4)
    plsc.subcore_barrier()

    def oslab(k, carry):
        sid = k * NUM_SUBCORES + s

        @pl.when(sid < NSLABS)
        def _():
            r0 = sid * SLAB
            pltpu.sync_copy(acc.at[pl.ds(r0, SLAB)],
                            rows_v[0].at[pl.ds(0, SLAB)])
            pltpu.sync_copy(rows_v[0].at[pl.ds(0, SLAB)],
                            out.at[c, pl.ds(r0, SLAB)])
        return carry
    lax.fori_loop(0, SITERS, oslab, 0)


CCHUNK = 200
NCCHUNKS = N_NODES // CCHUNK
CITERS = -(-NCCHUNKS // NW)


@functools.partial(
    pl.kernel,
    out_type=jax.ShapeDtypeStruct((N_NODES, DIM), jnp.float32),
    mesh=_mesh,
    scratch_types=[
        pltpu.VMEM((CCHUNK, DIM), jnp.float32),
        pltpu.VMEM((CCHUNK, DIM), jnp.float32),
    ],
)
def _combine(parts, out, buf_a, buf_b):
    c = lax.axis_index("c")
    s = lax.axis_index("s")
    wid = s * NUM_CORES + c

    def chunk(i, carry):
        cid = i * NW + wid

        @pl.when(cid < NCCHUNKS)
        def _():
            r0 = cid * CCHUNK
            pltpu.sync_copy(parts.at[0, pl.ds(r0, CCHUNK)], buf_a)
            pltpu.sync_copy(parts.at[1, pl.ds(r0, CCHUNK)], buf_b)
            _vec_add_rows(buf_a, buf_b, CCHUNK)
            pltpu.sync_copy(buf_a, out.at[pl.ds(r0, CCHUNK)])
        return carry
    lax.fori_loop(0, CITERS, chunk, 0)


BPW = BATCH // NW


@functools.partial(
    pl.kernel,
    out_type=(jax.ShapeDtypeStruct((BATCH, DIM), jnp.float32),
              jax.ShapeDtypeStruct((BATCH, DIM), jnp.float32),
              jax.ShapeDtypeStruct((BATCH, DIM), jnp.float32)),
    mesh=_mesh,
    scratch_types=[
        [pltpu.VMEM((BPW,), jnp.int32)] * 3,
        [pltpu.VMEM((BPW, DIM), jnp.float32)] * 4,
        [pltpu.SemaphoreType.DMA] * 4,
    ],
)
def _score_rows(t0, t1, t2a, t2b, ui, pi, ni, su, sp, sn,
                idx_v, gb, sem):
    c = lax.axis_index("c")
    s = lax.axis_index("s")
    wid = s * NUM_CORES + c
    base = wid * BPW

    tables = (t0, t1, t2a, t2b)
    for r, idx_hbm in enumerate((ui, pi, ni)):
        pltpu.sync_copy(idx_hbm.at[pl.ds(base, BPW)], idx_v[r])

    def issue_role(r):
        for t in range(4):
            pltpu.async_copy(tables[t].at[idx_v[r]], gb[t], sem[t])

    def wait_role(r):
        for t in range(4):
            pltpu.make_async_copy(tables[t].at[idx_v[r]], gb[t],
                                  sem[t]).wait()

    issue_role(0)
    for r, out in enumerate((su, sp, sn)):
        wait_role(r)

        def addrow(i, carry):
            for d in range(DIM // 16):
                sl = pl.ds(d * 16, 16)
                gb[0][i, sl] = ((gb[0][i, sl] + gb[1][i, sl])
                                + (gb[2][i, sl] + gb[3][i, sl]))
            return carry
        lax.fori_loop(0, BPW, addrow, 0)
        if r < 2:
            for t in range(1, 4):
                pltpu.async_copy(tables[t].at[idx_v[r + 1]], gb[t], sem[t])
        pltpu.sync_copy(gb[0], out.at[pl.ds(base, BPW)])
        if r < 2:
            pltpu.async_copy(tables[0].at[idx_v[r + 1]], gb[0], sem[0])


def _dots_tc(su_ref, sp_ref, sn_ref, pos_ref, neg_ref):
    inv9 = jnp.float32(1.0 / 9.0)
    su = su_ref[...]
    pos_ref[...] = jnp.sum(su * sp_ref[...], axis=1, keepdims=True) * inv9
    neg_ref[...] = jnp.sum(su * sn_ref[...], axis=1, keepdims=True) * inv9


_dots = pl.pallas_call(
    _dots_tc,
    out_shape=(jax.ShapeDtypeStruct((BATCH, 1), jnp.float32),
               jax.ShapeDtypeStruct((BATCH, 1), jnp.float32)),
)


def kernel(user_nodes, pos_item_nodes, neg_item_nodes, edge_index,
           edge_weight, emb_user, emb_item):
    emb0 = jnp.concatenate([emb_user, emb_item], axis=0)
    src = edge_index[0].astype(jnp.int32).reshape(NCHUNKS, ESUB, 128)
    dst = edge_index[1].astype(jnp.int32).reshape(NCHUNKS, ESUB, 128)
    ew = edge_weight.astype(jnp.float32).reshape(NCHUNKS, ECHUNK)

    p1 = _prop(emb0, src, dst, ew)
    emb1 = _combine(p1)
    p2 = _prop(emb1, src, dst, ew)
    su, sp, sn = _score_rows(emb0, emb1, p2[0], p2[1],
                             user_nodes.astype(jnp.int32),
                             pos_item_nodes.astype(jnp.int32),
                             neg_item_nodes.astype(jnp.int32))
    pos, neg = _dots(su, sp, sn)
    return pos[:, 0], neg[:, 0]


# speedup vs baseline: 1.1068x; 1.1068x over previous
import functools

import jax
import jax.numpy as jnp
from jax import lax
from jax.experimental import pallas as pl
from jax.experimental.pallas import tpu as pltpu
from jax.experimental.pallas import tpu_sc as plsc

N_NODES = 10000
DIM = 128
N_EDGES = 320000
BATCH = 4096

NUM_CORES = 2
NUM_SUBCORES = 16
NW = NUM_CORES * NUM_SUBCORES

ECHUNK = 128
ESUB = ECHUNK // 128
NCHUNKS = N_EDGES // ECHUNK
_EITERS_CEIL = -(-NCHUNKS // NW)
EITERS = _EITERS_CEIL + (_EITERS_CEIL % 2)

SLAB = 80
NSLABS = N_NODES // SLAB
SITERS = -(-NSLABS // NUM_SUBCORES)

_mesh = plsc.VectorSubcoreMesh(core_axis_name="c", subcore_axis_name="s")


def _vec_add_rows(dst_ref, src_ref, nrows):
    def body(r, carry):
        for d in range(DIM // 16):
            sl = pl.ds(d * 16, 16)
            dst_ref[r, sl] = dst_ref[r, sl] + src_ref[r, sl]
        return carry
    lax.fori_loop(0, nrows, body, 0)


@functools.partial(
    pl.kernel,
    out_type=jax.ShapeDtypeStruct((NUM_CORES, N_NODES, DIM), jnp.float32),
    mesh=_mesh,
    scratch_types=[
        [pltpu.VMEM((ESUB, 128), jnp.int32)] * 4,
        [pltpu.VMEM((ESUB, 128), jnp.int32)] * 4,
        [pltpu.VMEM((ECHUNK,), jnp.float32)] * 4,
        [pltpu.VMEM((ECHUNK, DIM), jnp.float32)] * 2,
        pltpu.VMEM_SHARED((N_NODES, DIM), jnp.float32),
        [pltpu.SemaphoreType.DMA] * 4,
        [pltpu.SemaphoreType.DMA] * 2,
        [pltpu.SemaphoreType.DMA] * 2,
    ],
)
def _prop(table, srcs, dsts, ews, out, src_v, dst_v, ew_v, rows_v,
          acc, semi, semg, sems):
    c = lax.axis_index("c")
    s = lax.axis_index("s")
    wid = s * NUM_CORES + c

    zero16 = jnp.zeros((16,), jnp.float32)

    def zrow(r, carry):
        for d in range(DIM // 16):
            rows_v[0][r, pl.ds(d * 16, 16)] = zero16
        return carry
    lax.fori_loop(0, ECHUNK, zrow, 0)

    def zslab(k, carry):
        sid = k * NUM_SUBCORES + s

        @pl.when(sid < NSLABS)
        def _():
            pltpu.sync_copy(rows_v[0].at[pl.ds(0, SLAB)],
                            acc.at[pl.ds(sid * SLAB, SLAB)])
        return carry
    lax.fori_loop(0, SITERS, zslab, 0)
    plsc.subcore_barrier()

    nchunks_j = jnp.int32(NCHUNKS)

    def valid(i):
        return i * NW + wid < nchunks_j

    def issue_idx(i, r):
        cid = jnp.minimum(i * NW + wid, nchunks_j - 1)
        pltpu.async_copy(srcs.at[cid], src_v[r], semi[r])
        pltpu.async_copy(dsts.at[cid], dst_v[r], semi[r])
        pltpu.async_copy(ews.at[cid], ew_v[r], semi[r])

    def wait_idx(r):
        pltpu.make_async_copy(srcs.at[0], src_v[r], semi[r]).wait()
        pltpu.make_async_copy(dsts.at[0], dst_v[r], semi[r]).wait()
        pltpu.make_async_copy(ews.at[0], ew_v[r], semi[r]).wait()

    def issue_gather(r, b):
        for j in range(ESUB):
            pltpu.async_copy(table.at[src_v[r].at[j]],
                             rows_v[b].at[pl.ds(j * 128, 128)], semg[b])

    def wait_gather(r, b):
        for j in range(ESUB):
            pltpu.make_async_copy(table.at[src_v[r].at[j]],
                                  rows_v[b].at[pl.ds(j * 128, 128)],
                                  semg[b]).wait()

    def issue_scatter(r, b):
        for j in range(ESUB):
            pltpu.async_copy(rows_v[b].at[pl.ds(j * 128, 128)],
                             acc.at[dst_v[r].at[j]], sems[b], add=True)

    def wait_scatter(r, b):
        for j in range(ESUB):
            pltpu.make_async_copy(rows_v[b].at[pl.ds(j * 128, 128)],
                                  acc.at[dst_v[r].at[j]], sems[b]).wait()

    for r in range(3):
        issue_idx(jnp.int32(r), r)
    wait_idx(0)
    issue_gather(0, 0)

    def pipe(k, carry):
        for u in range(4):
            i4 = k * 4 + u
            b = u % 2
            nb = 1 - b
            wait_gather(u, b)

            @pl.when((i4 >= 1) & valid(i4 - 1))
            def _():
                wait_scatter((u - 1) % 4, nb)
            wait_idx((u + 1) % 4)
            issue_gather((u + 1) % 4, nb)

            @pl.when(valid(i4))
            def _():
                def scale(g, carry2):
                    wv = ew_v[u][pl.ds(g * 16, 16)]
                    for j in range(16):
                        w = wv[j]
                        e = g * 16 + j
                        for d in range(DIM // 16):
                            sl = pl.ds(d * 16, 16)
                            rows_v[b][e, sl] = rows_v[b][e, sl] * w
                    return carry2
                lax.fori_loop(0, ECHUNK // 16, scale, 0)
                issue_scatter(u, b)
            issue_idx(i4 + 3, (u + 3) % 4)
        return carry
    lax.fori_loop(0, EITERS // 4, pipe, 0)
    wait_gather(EITERS % 4, EITERS % 2)

    @pl.when(valid(EITERS - 1))
    def _():
        wait_scatter((EITERS - 1) % 4, (EITERS - 1) % 2)
    wait_idx((EITERS + 1) % 4)
    wait_idx((EITERS + 2) % 4)
    plsc.subcore_barrier()

    def oslab(k, carry):
        sid = k * NUM_SUBCORES + s

        @pl.when(sid < NSLABS)
        def _():
            r0 = sid * SLAB
            pltpu.sync_copy(acc.at[pl.ds(r0, SLAB)],
                            rows_v[0].at[pl.ds(0, SLAB)])
            pltpu.sync_copy(rows_v[0].at[pl.ds(0, SLAB)],
                            out.at[c, pl.ds(r0, SLAB)])
        return carry
    lax.fori_loop(0, SITERS, oslab, 0)


CCHUNK = 200
NCCHUNKS = N_NODES // CCHUNK
CITERS = -(-NCCHUNKS // NW)


@functools.partial(
    pl.kernel,
    out_type=jax.ShapeDtypeStruct((N_NODES, DIM), jnp.float32),
    mesh=_mesh,
    scratch_types=[
        pltpu.VMEM((CCHUNK, DIM), jnp.float32),
        pltpu.VMEM((CCHUNK, DIM), jnp.float32),
    ],
)
def _combine(parts, out, buf_a, buf_b):
    c = lax.axis_index("c")
    s = lax.axis_index("s")
    wid = s * NUM_CORES + c

    def chunk(i, carry):
        cid = i * NW + wid

        @pl.when(cid < NCCHUNKS)
        def _():
            r0 = cid * CCHUNK
            pltpu.sync_copy(parts.at[0, pl.ds(r0, CCHUNK)], buf_a)
            pltpu.sync_copy(parts.at[1, pl.ds(r0, CCHUNK)], buf_b)
            _vec_add_rows(buf_a, buf_b, CCHUNK)
            pltpu.sync_copy(buf_a, out.at[pl.ds(r0, CCHUNK)])
        return carry
    lax.fori_loop(0, CITERS, chunk, 0)


BPW = BATCH // NW


@functools.partial(
    pl.kernel,
    out_type=(jax.ShapeDtypeStruct((BATCH, DIM), jnp.float32),
              jax.ShapeDtypeStruct((BATCH, DIM), jnp.float32),
              jax.ShapeDtypeStruct((BATCH, DIM), jnp.float32)),
    mesh=_mesh,
    scratch_types=[
        [pltpu.VMEM((BPW,), jnp.int32)] * 3,
        [pltpu.VMEM((BPW, DIM), jnp.float32)] * 4,
        [pltpu.SemaphoreType.DMA] * 4,
    ],
)
def _score_rows(t0, t1, t2a, t2b, ui, pi, ni, su, sp, sn,
                idx_v, gb, sem):
    c = lax.axis_index("c")
    s = lax.axis_index("s")
    wid = s * NUM_CORES + c
    base = wid * BPW

    tables = (t0, t1, t2a, t2b)
    for r, idx_hbm in enumerate((ui, pi, ni)):
        pltpu.sync_copy(idx_hbm.at[pl.ds(base, BPW)], idx_v[r])

    def issue_role(r):
        for t in range(4):
            pltpu.async_copy(tables[t].at[idx_v[r]], gb[t], sem[t])

    def wait_role(r):
        for t in range(4):
            pltpu.make_async_copy(tables[t].at[idx_v[r]], gb[t],
                                  sem[t]).wait()

    issue_role(0)
    for r, out in enumerate((su, sp, sn)):
        wait_role(r)

        def addrow(i, carry):
            for d in range(DIM // 16):
                sl = pl.ds(d * 16, 16)
                gb[0][i, sl] = ((gb[0][i, sl] + gb[1][i, sl])
                                + (gb[2][i, sl] + gb[3][i, sl]))
            return carry
        lax.fori_loop(0, BPW, addrow, 0)
        if r < 2:
            for t in range(1, 4):
                pltpu.async_copy(tables[t].at[idx_v[r + 1]], gb[t], sem[t])
        pltpu.sync_copy(gb[0], out.at[pl.ds(base, BPW)])
        if r < 2:
            pltpu.async_copy(tables[0].at[idx_v[r + 1]], gb[0], sem[0])


def _dots_tc(su_ref, sp_ref, sn_ref, pos_ref, neg_ref):
    inv9 = jnp.float32(1.0 / 9.0)
    su = su_ref[...]
    pos_ref[...] = jnp.sum(su * sp_ref[...], axis=1, keepdims=True) * inv9
    neg_ref[...] = jnp.sum(su * sn_ref[...], axis=1, keepdims=True) * inv9


_dots = pl.pallas_call(
    _dots_tc,
    out_shape=(jax.ShapeDtypeStruct((BATCH, 1), jnp.float32),
               jax.ShapeDtypeStruct((BATCH, 1), jnp.float32)),
)


def kernel(user_nodes, pos_item_nodes, neg_item_nodes, edge_index,
           edge_weight, emb_user, emb_item):
    emb0 = jnp.concatenate([emb_user, emb_item], axis=0)
    src = edge_index[0].astype(jnp.int32).reshape(NCHUNKS, ESUB, 128)
    dst = edge_index[1].astype(jnp.int32).reshape(NCHUNKS, ESUB, 128)
    ew = edge_weight.astype(jnp.float32).reshape(NCHUNKS, ECHUNK)

    p1 = _prop(emb0, src, dst, ew)
    emb1 = _combine(p1)
    p2 = _prop(emb1, src, dst, ew)
    su, sp, sn = _score_rows(emb0, emb1, p2[0], p2[1],
                             user_nodes.astype(jnp.int32),
                             pos_item_nodes.astype(jnp.int32),
                             neg_item_nodes.astype(jnp.int32))
    pos, neg = _dots(su, sp, sn)
    return pos[:, 0], neg[:, 0]


# speedup vs baseline: 1.1468x; 1.0362x over previous
import functools

import jax
import jax.numpy as jnp
from jax import lax
from jax.experimental import pallas as pl
from jax.experimental.pallas import tpu as pltpu
from jax.experimental.pallas import tpu_sc as plsc

N_NODES = 10000
DIM = 128
N_EDGES = 320000
BATCH = 4096

NUM_CORES = 2
NUM_SUBCORES = 16
NW = NUM_CORES * NUM_SUBCORES

ECHUNK = 128
ESUB = ECHUNK // 128
NCHUNKS = N_EDGES // ECHUNK
_EITERS_CEIL = -(-NCHUNKS // NW)
EITERS = _EITERS_CEIL + (_EITERS_CEIL % 2)

SLAB = 80
NSLABS = N_NODES // SLAB
SITERS = -(-NSLABS // NUM_SUBCORES)

_mesh = plsc.VectorSubcoreMesh(core_axis_name="c", subcore_axis_name="s")


def _vec_add_rows(dst_ref, src_ref, nrows):
    def body(r, carry):
        for d in range(DIM // 16):
            sl = pl.ds(d * 16, 16)
            dst_ref[r, sl] = dst_ref[r, sl] + src_ref[r, sl]
        return carry
    lax.fori_loop(0, nrows, body, 0)


@functools.partial(
    pl.kernel,
    out_type=jax.ShapeDtypeStruct((NUM_CORES, N_NODES, DIM), jnp.float32),
    mesh=_mesh,
    scratch_types=[
        [pltpu.VMEM((ESUB, 128), jnp.int32)] * 4,
        [pltpu.VMEM((ESUB, 128), jnp.int32)] * 4,
        [pltpu.VMEM((ECHUNK,), jnp.float32)] * 4,
        [pltpu.VMEM((ECHUNK, DIM), jnp.float32)] * 2,
        pltpu.VMEM_SHARED((N_NODES, DIM), jnp.float32),
        [pltpu.SemaphoreType.DMA] * 4,
        [pltpu.SemaphoreType.DMA] * 2,
        [pltpu.SemaphoreType.DMA] * 2,
    ],
)
def _prop(table, srcs, dsts, ews, out, src_v, dst_v, ew_v, rows_v,
          acc, semi, semg, sems):
    c = lax.axis_index("c")
    s = lax.axis_index("s")
    wid = s * NUM_CORES + c

    zero16 = jnp.zeros((16,), jnp.float32)

    def zrow(r, carry):
        for d in range(DIM // 16):
            rows_v[0][r, pl.ds(d * 16, 16)] = zero16
        return carry
    lax.fori_loop(0, ECHUNK, zrow, 0)

    def zslab(k, carry):
        sid = k * NUM_SUBCORES + s

        @pl.when(sid < NSLABS)
        def _():
            pltpu.sync_copy(rows_v[0].at[pl.ds(0, SLAB)],
                            acc.at[pl.ds(sid * SLAB, SLAB)])
        return carry
    lax.fori_loop(0, SITERS, zslab, 0)
    plsc.subcore_barrier()

    nchunks_j = jnp.int32(NCHUNKS)

    def valid(i):
        return i * NW + wid < nchunks_j

    def issue_idx(i, r):
        cid = jnp.minimum(i * NW + wid, nchunks_j - 1)
        pltpu.async_copy(srcs.at[cid], src_v[r], semi[r])
        pltpu.async_copy(dsts.at[cid], dst_v[r], semi[r])
        pltpu.async_copy(ews.at[cid], ew_v[r], semi[r])

    def wait_idx(r):
        pltpu.make_async_copy(srcs.at[0], src_v[r], semi[r]).wait()
        pltpu.make_async_copy(dsts.at[0], dst_v[r], semi[r]).wait()
        pltpu.make_async_copy(ews.at[0], ew_v[r], semi[r]).wait()

    def issue_gather(r, b):
        for j in range(ESUB):
            pltpu.async_copy(table.at[src_v[r].at[j]],
                             rows_v[b].at[pl.ds(j * 128, 128)], semg[b])

    def wait_gather(r, b):
        for j in range(ESUB):
            pltpu.make_async_copy(table.at[src_v[r].at[j]],
                                  rows_v[b].at[pl.ds(j * 128, 128)],
                                  semg[b]).wait()

    def issue_scatter(r, b):
        for j in range(ESUB):
            pltpu.async_copy(rows_v[b].at[pl.ds(j * 128, 128)],
                             acc.at[dst_v[r].at[j]], sems[b], add=True)

    def wait_scatter(r, b):
        for j in range(ESUB):
            pltpu.make_async_copy(rows_v[b].at[pl.ds(j * 128, 128)],
                                  acc.at[dst_v[r].at[j]], sems[b]).wait()

    for r in range(3):
        issue_idx(jnp.int32(r), r)
    wait_idx(0)
    issue_gather(0, 0)

    def pipe(k, carry):
        for u in range(4):
            i4 = k * 4 + u
            b = u % 2
            nb = 1 - b
            wait_gather(u, b)

            @pl.when((i4 >= 1) & valid(i4 - 1))
            def _():
                wait_scatter((u - 1) % 4, nb)
            wait_idx((u + 1) % 4)
            issue_gather((u + 1) % 4, nb)

            @pl.when(valid(i4))
            def _():
                def scale(g, carry2):
                    wv = ew_v[u][pl.ds(g * 16, 16)]
                    for j in range(16):
                        w = wv[j]
                        e = g * 16 + j
                        for d in range(DIM // 16):
                            sl = pl.ds(d * 16, 16)
                            rows_v[b][e, sl] = rows_v[b][e, sl] * w
                    return carry2
                lax.fori_loop(0, ECHUNK // 16, scale, 0)
                issue_scatter(u, b)
            issue_idx(i4 + 3, (u + 3) % 4)
        return carry
    lax.fori_loop(0, EITERS // 4, pipe, 0)
    wait_gather(EITERS % 4, EITERS % 2)

    @pl.when(valid(EITERS - 1))
    def _():
        wait_scatter((EITERS - 1) % 4, (EITERS - 1) % 2)
    wait_idx((EITERS + 1) % 4)
    wait_idx((EITERS + 2) % 4)
    plsc.subcore_barrier()

    def oslab(k, carry):
        sid = k * NUM_SUBCORES + s

        @pl.when(sid < NSLABS)
        def _():
            r0 = sid * SLAB
            pltpu.sync_copy(acc.at[pl.ds(r0, SLAB)],
                            rows_v[0].at[pl.ds(0, SLAB)])
            pltpu.sync_copy(rows_v[0].at[pl.ds(0, SLAB)],
                            out.at[c, pl.ds(r0, SLAB)])
        return carry
    lax.fori_loop(0, SITERS, oslab, 0)


def _combine_tc_body(parts_ref, out_ref):
    out_ref[...] = parts_ref[0] + parts_ref[1]


_combine = pl.pallas_call(
    _combine_tc_body,
    out_shape=jax.ShapeDtypeStruct((N_NODES, DIM), jnp.float32),
)


BPW = BATCH // NW


@functools.partial(
    pl.kernel,
    out_type=(jax.ShapeDtypeStruct((BATCH, DIM), jnp.float32),
              jax.ShapeDtypeStruct((BATCH, DIM), jnp.float32),
              jax.ShapeDtypeStruct((BATCH, DIM), jnp.float32)),
    mesh=_mesh,
    scratch_types=[
        [pltpu.VMEM((BPW,), jnp.int32)] * 3,
        [pltpu.VMEM((BPW, DIM), jnp.float32)] * 4,
        [pltpu.SemaphoreType.DMA] * 4,
    ],
)
def _score_rows(t0, t1, t2a, t2b, ui, pi, ni, su, sp, sn,
                idx_v, gb, sem):
    c = lax.axis_index("c")
    s = lax.axis_index("s")
    wid = s * NUM_CORES + c
    base = wid * BPW

    tables = (t0, t1, t2a, t2b)
    for r, idx_hbm in enumerate((ui, pi, ni)):
        pltpu.sync_copy(idx_hbm.at[pl.ds(base, BPW)], idx_v[r])

    def issue_role(r):
        for t in range(4):
            pltpu.async_copy(tables[t].at[idx_v[r]], gb[t], sem[t])

    def wait_role(r):
        for t in range(4):
            pltpu.make_async_copy(tables[t].at[idx_v[r]], gb[t],
                                  sem[t]).wait()

    issue_role(0)
    for r, out in enumerate((su, sp, sn)):
        wait_role(r)

        def addrow(i, carry):
            for d in range(DIM // 16):
                sl = pl.ds(d * 16, 16)
                gb[0][i, sl] = ((gb[0][i, sl] + gb[1][i, sl])
                                + (gb[2][i, sl] + gb[3][i, sl]))
            return carry
        lax.fori_loop(0, BPW, addrow, 0)
        if r < 2:
            for t in range(1, 4):
                pltpu.async_copy(tables[t].at[idx_v[r + 1]], gb[t], sem[t])
        pltpu.sync_copy(gb[0], out.at[pl.ds(base, BPW)])
        if r < 2:
            pltpu.async_copy(tables[0].at[idx_v[r + 1]], gb[0], sem[0])


def _dots_tc(su_ref, sp_ref, sn_ref, pos_ref, neg_ref):
    inv9 = jnp.float32(1.0 / 9.0)
    su = su_ref[...]
    pos_ref[...] = jnp.sum(su * sp_ref[...], axis=1, keepdims=True) * inv9
    neg_ref[...] = jnp.sum(su * sn_ref[...], axis=1, keepdims=True) * inv9


_dots = pl.pallas_call(
    _dots_tc,
    out_shape=(jax.ShapeDtypeStruct((BATCH, 1), jnp.float32),
               jax.ShapeDtypeStruct((BATCH, 1), jnp.float32)),
)


def kernel(user_nodes, pos_item_nodes, neg_item_nodes, edge_index,
           edge_weight, emb_user, emb_item):
    emb0 = jnp.concatenate([emb_user, emb_item], axis=0)
    src = edge_index[0].astype(jnp.int32).reshape(NCHUNKS, ESUB, 128)
    dst = edge_index[1].astype(jnp.int32).reshape(NCHUNKS, ESUB, 128)
    ew = edge_weight.astype(jnp.float32).reshape(NCHUNKS, ECHUNK)

    p1 = _prop(emb0, src, dst, ew)
    emb1 = _combine(p1)
    p2 = _prop(emb1, src, dst, ew)
    su, sp, sn = _score_rows(emb0, emb1, p2[0], p2[1],
                             user_nodes.astype(jnp.int32),
                             pos_item_nodes.astype(jnp.int32),
                             neg_item_nodes.astype(jnp.int32))
    pos, neg = _dots(su, sp, sn)
    return pos[:, 0], neg[:, 0]
